# baseline (speedup 1.0000x reference)
import functools

import jax
import jax.numpy as jnp
from jax import lax
from jax.experimental import pallas as pl
from jax.experimental.pallas import tpu as pltpu
from jax.experimental.pallas import tpu_sc as plsc

B = 16384
F_SP = 26
F_DN = 13
V = 100000

NC = 2
NS = 16
NW = NC * NS
BPW = B // NW
L = 16
CHUNK = 128
NCH = BPW // CHUNK
NR = F_SP * NCH


NSEM = 4


def _body(idx_hbm, dense_hbm, w_hbm, table_hbm, out_hbm,
          idx_v, vals_v, dense_v, w_v, out_v, *sems):
    cid = lax.axis_index("c")
    sid = lax.axis_index("s")
    wid = sid * NC + cid
    base = wid * BPW

    pltpu.sync_copy(idx_hbm.at[wid], idx_v)
    pltpu.sync_copy(dense_hbm.at[wid], dense_v)
    pltpu.sync_copy(w_hbm, w_v)

    copies = [[] for _ in range(NCH)]
    for c in range(NCH):
        for f in range(F_SP):
            r = f * NCH + c
            off = f * V
            for j in range(CHUNK // L):
                s = j * L
                idx_v[r, 0, pl.ds(s, L)] = idx_v[r, 0, pl.ds(s, L)] + off
        for f in range(F_SP):
            r = f * NCH + c
            copies[c].append(
                pltpu.async_copy(
                    table_hbm.at[idx_v.at[r]],
                    vals_v.at[r],
                    sems[c],
                )
            )

    for c in range(NCH):
        for cp in copies[c]:
            cp.wait()

        for j in range(CHUNK // L):
            s = j * L
            terms = [vals_v[f * NCH + c, 0, pl.ds(s, L)] for f in range(F_SP)]
            terms += [
                dense_v[f, pl.ds(c * CHUNK + s, L)] * w_v[f]
                for f in range(F_DN)
            ]
            while len(terms) > 1:
                nxt = [terms[i] + terms[i + 1]
                       for i in range(0, len(terms) - 1, 2)]
                if len(terms) % 2:
                    nxt.append(terms[-1])
                terms = nxt
            out_v[pl.ds(c * CHUNK + s, L)] = 1.0 / (1.0 + jnp.exp(-terms[0]))

    pltpu.sync_copy(out_v, out_hbm.at[pl.ds(base, BPW)])


@jax.jit
def _sc_call(idx_t, dense_t, w_b, table):
    run = pl.kernel(
        _body,
        out_type=jax.ShapeDtypeStruct((B,), jnp.float32),
        mesh=plsc.VectorSubcoreMesh(core_axis_name="c", subcore_axis_name="s"),
        scratch_types=[
            pltpu.VMEM((NR, 1, CHUNK), jnp.int32),
            pltpu.VMEM((NR, 1, CHUNK), jnp.float32),
            pltpu.VMEM((F_DN, BPW), jnp.float32),
            pltpu.VMEM((F_DN, L), jnp.float32),
            pltpu.VMEM((BPW,), jnp.float32),
        ] + [pltpu.SemaphoreType.DMA] * NSEM,
    )
    return run(idx_t, dense_t, w_b, table)


def kernel(sparse_idx, dense_vals, lin_table, dense_w):
    idx_t = (
        sparse_idx.reshape(NW, NCH, CHUNK, F_SP)
        .transpose(0, 3, 1, 2)
        .reshape(NW, NR, 1, CHUNK)
    )
    dense_t = dense_vals.reshape(NW, BPW, F_DN).transpose(0, 2, 1)
    w_b = jnp.broadcast_to(dense_w.reshape(F_DN, 1), (F_DN, L))
    out = _sc_call(idx_t, dense_t, w_b, lin_table.reshape(1, -1))
    return out.reshape(B, 1)


# speedup vs baseline: 1.0798x; 1.0798x over previous
import functools

import jax
import jax.numpy as jnp
from jax import lax
from jax.experimental import pallas as pl
from jax.experimental.pallas import tpu as pltpu
from jax.experimental.pallas import tpu_sc as plsc

B = 16384
F_SP = 26
F_DN = 13
V = 100000

NC = 2
NS = 16
NW = NC * NS
BPW = B // NW
L = 16
CHUNK = 128
NCH = BPW // CHUNK
NR = F_SP * NCH


NSEM = 4


def _body(idx_hbm, dense_hbm, w_hbm, table_hbm, out_hbm,
          idx_v, vals_v, dense_v, w_v, out_v, *sems):
    cid = lax.axis_index("c")
    sid = lax.axis_index("s")
    wid = sid * NC + cid
    base = wid * BPW

    pltpu.sync_copy(idx_hbm.at[wid], idx_v)
    pltpu.sync_copy(dense_hbm.at[wid], dense_v)
    pltpu.sync_copy(w_hbm, w_v)

    copies = [[] for _ in range(NCH)]
    for c in range(NCH):
        for f in range(F_SP):
            r = f * NCH + c
            off = f * V

            def _obody(j, _, r=r, off=off):
                s = j * L
                idx_v[r, 0, pl.ds(s, L)] = idx_v[r, 0, pl.ds(s, L)] + off
                return 0

            lax.fori_loop(0, CHUNK // L, _obody, 0)
        for f in range(F_SP):
            r = f * NCH + c
            copies[c].append(
                pltpu.async_copy(
                    table_hbm.at[idx_v.at[r]],
                    vals_v.at[r],
                    sems[c],
                )
            )

    for c in range(NCH):
        for cp in copies[c]:
            cp.wait()

        def _cbody(j, _, c=c):
            s = j * L
            acc = vals_v[c, 0, pl.ds(s, L)]
            for f in range(1, F_SP):
                acc = acc + vals_v[f * NCH + c, 0, pl.ds(s, L)]
            for f in range(F_DN):
                acc = acc + dense_v[f, pl.ds(c * CHUNK + s, L)] * w_v[f]
            out_v[pl.ds(c * CHUNK + s, L)] = 1.0 / (1.0 + jnp.exp(-acc))
            return 0

        lax.fori_loop(0, CHUNK // L, _cbody, 0)

    pltpu.sync_copy(out_v, out_hbm.at[pl.ds(base, BPW)])


@jax.jit
def _sc_call(idx_t, dense_t, w_b, table):
    run = pl.kernel(
        _body,
        out_type=jax.ShapeDtypeStruct((B,), jnp.float32),
        mesh=plsc.VectorSubcoreMesh(core_axis_name="c", subcore_axis_name="s"),
        scratch_types=[
            pltpu.VMEM((NR, 1, CHUNK), jnp.int32),
            pltpu.VMEM((NR, 1, CHUNK), jnp.float32),
            pltpu.VMEM((F_DN, BPW), jnp.float32),
            pltpu.VMEM((F_DN, L), jnp.float32),
            pltpu.VMEM((BPW,), jnp.float32),
        ] + [pltpu.SemaphoreType.DMA] * NSEM,
    )
    return run(idx_t, dense_t, w_b, table)


def kernel(sparse_idx, dense_vals, lin_table, dense_w):
    idx_t = (
        sparse_idx.reshape(NW, NCH, CHUNK, F_SP)
        .transpose(0, 3, 1, 2)
        .reshape(NW, NR, 1, CHUNK)
    )
    dense_t = dense_vals.reshape(NW, BPW, F_DN).transpose(0, 2, 1)
    w_b = jnp.broadcast_to(dense_w.reshape(F_DN, 1), (F_DN, L))
    out = _sc_call(idx_t, dense_t, w_b, lin_table.reshape(1, -1))
    return out.reshape(B, 1)


# speedup vs baseline: 1.1042x; 1.0226x over previous
import functools

import jax
import jax.numpy as jnp
from jax import lax
from jax.experimental import pallas as pl
from jax.experimental.pallas import tpu as pltpu
from jax.experimental.pallas import tpu_sc as plsc

B = 16384
F_SP = 26
F_DN = 13
V = 100000

NC = 2
NS = 16
NW = NC * NS
BPW = B // NW
L = 16
CHUNK = 128
NCH = BPW // CHUNK
NR = F_SP * NCH


NSEM = 5


def _body(idx_hbm, dense_hbm, w_hbm, table_hbm, out_hbm,
          idx_v, vals_v, dense_v, w_v, out_v, *sems):
    cid = lax.axis_index("c")
    sid = lax.axis_index("s")
    wid = sid * NC + cid
    base = wid * BPW

    pltpu.sync_copy(idx_hbm.at[wid], idx_v)
    dense_cp = pltpu.async_copy(dense_hbm.at[wid], dense_v, sems[NCH])
    w_cp = pltpu.async_copy(w_hbm, w_v, sems[NCH])

    copies = [[] for _ in range(NCH)]
    for c in range(NCH):
        for f in range(F_SP):
            r = f * NCH + c
            off = f * V

            def _obody(j, _, r=r, off=off):
                s = j * L
                idx_v[r, 0, pl.ds(s, L)] = idx_v[r, 0, pl.ds(s, L)] + off
                return 0

            lax.fori_loop(0, CHUNK // L, _obody, 0)
        for f in range(F_SP):
            r = f * NCH + c
            copies[c].append(
                pltpu.async_copy(
                    table_hbm.at[idx_v.at[r]],
                    vals_v.at[r],
                    sems[c],
                )
            )

    dense_cp.wait()
    w_cp.wait()
    for c in range(NCH):
        for cp in copies[c]:
            cp.wait()

        def _cbody(j, _, c=c):
            s = j * L
            acc = vals_v[c, 0, pl.ds(s, L)]
            for f in range(1, F_SP):
                acc = acc + vals_v[f * NCH + c, 0, pl.ds(s, L)]
            for f in range(F_DN):
                acc = acc + dense_v[f, pl.ds(c * CHUNK + s, L)] * w_v[f]
            out_v[pl.ds(c * CHUNK + s, L)] = 1.0 / (1.0 + jnp.exp(-acc))
            return 0

        lax.fori_loop(0, CHUNK // L, _cbody, 0)

    pltpu.sync_copy(out_v, out_hbm.at[pl.ds(base, BPW)])


@jax.jit
def _sc_call(idx_t, dense_t, w_b, table):
    run = pl.kernel(
        _body,
        out_type=jax.ShapeDtypeStruct((B,), jnp.float32),
        mesh=plsc.VectorSubcoreMesh(core_axis_name="c", subcore_axis_name="s"),
        scratch_types=[
            pltpu.VMEM((NR, 1, CHUNK), jnp.int32),
            pltpu.VMEM((NR, 1, CHUNK), jnp.float32),
            pltpu.VMEM((F_DN, BPW), jnp.float32),
            pltpu.VMEM((F_DN, L), jnp.float32),
            pltpu.VMEM((BPW,), jnp.float32),
        ] + [pltpu.SemaphoreType.DMA] * NSEM,
    )
    return run(idx_t, dense_t, w_b, table)


def kernel(sparse_idx, dense_vals, lin_table, dense_w):
    idx_t = (
        sparse_idx.reshape(NW, NCH, CHUNK, F_SP)
        .transpose(0, 3, 1, 2)
        .reshape(NW, NR, 1, CHUNK)
    )
    dense_t = dense_vals.reshape(NW, BPW, F_DN).transpose(0, 2, 1)
    w_b = jnp.broadcast_to(dense_w.reshape(F_DN, 1), (F_DN, L))
    out = _sc_call(idx_t, dense_t, w_b, lin_table.reshape(1, -1))
    return out.reshape(B, 1)
